# baseline (speedup 1.0000x reference)
import math

import jax
import jax.numpy as jnp
from jax import lax
from jax.experimental import pallas as pl
from jax.experimental.pallas import tpu as pltpu
from jax.experimental.pallas import tpu_sc as plsc

_EMB = 32
_SCALE = math.sqrt(_EMB)
_NC, _NS = 2, 16
_NW = _NC * _NS
_SUB = 8
_LANE = 128


def _body(idx_hbm, table_hbm, out_hbm, idx_v, rows_v, trans_v, gsem, osem):
    wid = lax.axis_index("s") * _NC + lax.axis_index("c")
    steps = idx_hbm.shape[0]

    iota = lax.iota(jnp.int32, 16)
    bvecs = [iota + 16 * k for k in range(8)]
    evecs = [jnp.full((16,), e, jnp.int32) for e in range(_EMB)]

    def load_idx(g, b):
        pltpu.sync_copy(idx_hbm.at[g, wid], idx_v.at[b])

    def fire_gather(b):
        for s in range(_SUB):
            pltpu.async_copy(table_hbm.at[idx_v.at[b, s]],
                             rows_v.at[b, s], gsem)

    def out_desc(g):
        return pltpu.make_async_copy(
            trans_v, out_hbm.at[pl.ds(_SUB * g, _SUB), :, wid], osem)

    def process(g, b):
        @pl.when(g > 0)
        def _():
            out_desc(g - 1).wait()

        @pl.loop(0, _SUB)
        def _(s):
            pltpu.make_async_copy(table_hbm.at[idx_v.at[b, s]],
                                  rows_v.at[b, s], gsem).wait()
            rows = rows_v.at[b, s]
            for eg in range(4):
                for ei in range(8):
                    e = 8 * eg + ei
                    for k in range(8):
                        v = plsc.load_gather(rows, [bvecs[k], evecs[e]])
                        trans_v[s, eg, ei, pl.ds(16 * k, 16)] = v * _SCALE
        out_desc(g).start()

    load_idx(0, 0)
    fire_gather(0)

    @pl.loop(0, (steps + 1) // 2)
    def _(t):
        g0 = 2 * t
        @pl.when(g0 + 1 < steps)
        def _():
            load_idx(g0 + 1, 1)
            fire_gather(1)
        process(g0, 0)

        @pl.when(g0 + 1 < steps)
        def _():
            @pl.when(g0 + 2 < steps)
            def _():
                load_idx(g0 + 2, 0)
                fire_gather(0)
            process(g0 + 1, 1)

    out_desc(steps - 1).wait()


@jax.jit
def _gather(idx4d, table):
    na = idx4d.shape[0]
    mesh = plsc.VectorSubcoreMesh(core_axis_name="c", subcore_axis_name="s",
                                  num_cores=_NC, num_subcores=_NS)
    f = pl.kernel(
        _body,
        out_type=jax.ShapeDtypeStruct(
            (na * _SUB, _EMB // 8, _NW, 8, _LANE), jnp.float32),
        mesh=mesh,
        compiler_params=pltpu.CompilerParams(use_tc_tiling_on_sc=False,
                                             needs_layout_passes=False),
        scratch_types=[
            pltpu.VMEM((2, _SUB, _LANE), jnp.int32),
            pltpu.VMEM((2, _SUB, _LANE, _EMB), jnp.float32),
            pltpu.VMEM((_SUB, _EMB // 8, 8, _LANE), jnp.float32),
            pltpu.SemaphoreType.DMA,
            pltpu.SemaphoreType.DMA,
        ],
    )
    return f(idx4d, table)


def kernel(tokens, table):
    bsz, seq = tokens.shape
    na, nb = seq // _SUB, bsz // _LANE
    tv = (jnp.transpose(tokens.astype(jnp.int32))
          .reshape(na, _SUB, nb, _LANE).transpose(0, 2, 1, 3))
    lin = _gather(tv, table)
    return lin.transpose(2, 4, 0, 1, 3).reshape(bsz, seq, _EMB)


# speedup vs baseline: 1.2425x; 1.2425x over previous
import math

import jax
import jax.numpy as jnp
from jax import lax
from jax.experimental import pallas as pl
from jax.experimental.pallas import tpu as pltpu
from jax.experimental.pallas import tpu_sc as plsc

_EMB = 32
_SCALE = math.sqrt(_EMB)
_NC, _NS = 2, 16
_NW = _NC * _NS
_SUB = 8
_LANE = 128


def _body(idx_hbm, table_hbm, out_hbm, idx_v, rows_v, trans_v, gsem, osem):
    wid = lax.axis_index("s") * _NC + lax.axis_index("c")
    steps = idx_hbm.shape[0]

    iota = lax.iota(jnp.int32, 16)

    def load_idx(g, b):
        pltpu.sync_copy(idx_hbm.at[g, wid], idx_v.at[b])

    def fire_gather(b):
        for s in range(_SUB):
            pltpu.async_copy(table_hbm.at[idx_v.at[b, s]],
                             rows_v.at[b, s], gsem)

    def out_desc(g):
        return pltpu.make_async_copy(
            trans_v, out_hbm.at[pl.ds(_SUB * g, _SUB), :, wid], osem)

    def process(g, b):
        @pl.when(g > 0)
        def _():
            out_desc(g - 1).wait()

        @pl.loop(0, _SUB)
        def _(s):
            pltpu.make_async_copy(table_hbm.at[idx_v.at[b, s]],
                                  rows_v.at[b, s], gsem).wait()
            rows = rows_v.at[b, s]

            @plsc.parallel_loop(0, _EMB * 8, unroll=4)
            def _(i):
                e = i >> 3
                k = i & 7
                bvec = iota + k * 16
                evec = jnp.broadcast_to(e, (16,))
                v = plsc.load_gather(rows, [bvec, evec])
                trans_v[s, i >> 6, (i >> 3) & 7,
                        pl.ds((i & 7) * 16, 16)] = v * _SCALE
        out_desc(g).start()

    load_idx(0, 0)
    fire_gather(0)

    @pl.loop(0, (steps + 1) // 2)
    def _(t):
        g0 = 2 * t
        @pl.when(g0 + 1 < steps)
        def _():
            load_idx(g0 + 1, 1)
            fire_gather(1)
        process(g0, 0)

        @pl.when(g0 + 1 < steps)
        def _():
            @pl.when(g0 + 2 < steps)
            def _():
                load_idx(g0 + 2, 0)
                fire_gather(0)
            process(g0 + 1, 1)

    out_desc(steps - 1).wait()


@jax.jit
def _gather(idx4d, table):
    na = idx4d.shape[0]
    mesh = plsc.VectorSubcoreMesh(core_axis_name="c", subcore_axis_name="s",
                                  num_cores=_NC, num_subcores=_NS)
    f = pl.kernel(
        _body,
        out_type=jax.ShapeDtypeStruct(
            (na * _SUB, _EMB // 8, _NW, 8, _LANE), jnp.float32),
        mesh=mesh,
        compiler_params=pltpu.CompilerParams(use_tc_tiling_on_sc=False,
                                             needs_layout_passes=False),
        scratch_types=[
            pltpu.VMEM((2, _SUB, _LANE), jnp.int32),
            pltpu.VMEM((2, _SUB, _LANE, _EMB), jnp.float32),
            pltpu.VMEM((_SUB, _EMB // 8, 8, _LANE), jnp.float32),
            pltpu.SemaphoreType.DMA,
            pltpu.SemaphoreType.DMA,
        ],
    )
    return f(idx4d, table)


def kernel(tokens, table):
    bsz, seq = tokens.shape
    na, nb = seq // _SUB, bsz // _LANE
    tv = (jnp.transpose(tokens.astype(jnp.int32))
          .reshape(na, _SUB, nb, _LANE).transpose(0, 2, 1, 3))
    lin = _gather(tv, table)
    return lin.transpose(2, 4, 0, 1, 3).reshape(bsz, seq, _EMB)


# speedup vs baseline: 2.0718x; 1.6674x over previous
import math

import jax
import jax.numpy as jnp
from jax import lax
from jax.experimental import pallas as pl
from jax.experimental.pallas import tpu as pltpu
from jax.experimental.pallas import tpu_sc as plsc

_EMB = 32
_SCALE = math.sqrt(_EMB)
_NC, _NS = 2, 16
_NW = _NC * _NS
_SUB = 8
_LANE = 128


def _body(idx_hbm, table_hbm, out_hbm, idx_v, rows_v, trans_v, gsem, osem):
    wid = lax.axis_index("s") * _NC + lax.axis_index("c")
    steps = idx_hbm.shape[0]

    iota = lax.iota(jnp.int32, 16)

    def load_idx(g, b):
        pltpu.sync_copy(idx_hbm.at[g, wid], idx_v.at[b])

    def fire_gather(b):
        for s in range(_SUB):
            pltpu.async_copy(table_hbm.at[idx_v.at[b, s]],
                             rows_v.at[b, s], gsem)

    def out_desc(g):
        return pltpu.make_async_copy(
            trans_v, out_hbm.at[pl.ds(_SUB * g, _SUB), :, wid], osem)

    def process(g, b):
        @pl.when(g > 0)
        def _():
            out_desc(g - 1).wait()

        @pl.loop(0, _SUB)
        def _(s):
            pltpu.make_async_copy(table_hbm.at[idx_v.at[b, s]],
                                  rows_v.at[b, s], gsem).wait()
            rows = rows_v.at[b, s]

            @plsc.parallel_loop(0, _EMB * 8, unroll=4)
            def _(i):
                b0 = (i >> 5) * 16
                e0 = i & 31
                bvec = iota + b0
                evec = (iota + e0) & 31
                v = plsc.load_gather(rows, [bvec, evec])
                plsc.store_scatter(trans_v.at[s],
                                   [evec >> 3, evec & 7, bvec],
                                   v * _SCALE)
        out_desc(g).start()

    load_idx(0, 0)
    fire_gather(0)

    @pl.loop(0, (steps + 1) // 2)
    def _(t):
        g0 = 2 * t
        @pl.when(g0 + 1 < steps)
        def _():
            load_idx(g0 + 1, 1)
            fire_gather(1)
        process(g0, 0)

        @pl.when(g0 + 1 < steps)
        def _():
            @pl.when(g0 + 2 < steps)
            def _():
                load_idx(g0 + 2, 0)
                fire_gather(0)
            process(g0 + 1, 1)

    out_desc(steps - 1).wait()


@jax.jit
def _gather(idx4d, table):
    na = idx4d.shape[0]
    mesh = plsc.VectorSubcoreMesh(core_axis_name="c", subcore_axis_name="s",
                                  num_cores=_NC, num_subcores=_NS)
    f = pl.kernel(
        _body,
        out_type=jax.ShapeDtypeStruct(
            (na * _SUB, _EMB // 8, _NW, 8, _LANE), jnp.float32),
        mesh=mesh,
        compiler_params=pltpu.CompilerParams(use_tc_tiling_on_sc=False,
                                             needs_layout_passes=False),
        scratch_types=[
            pltpu.VMEM((2, _SUB, _LANE), jnp.int32),
            pltpu.VMEM((2, _SUB, _LANE, _EMB), jnp.float32),
            pltpu.VMEM((_SUB, _EMB // 8, 8, _LANE), jnp.float32),
            pltpu.SemaphoreType.DMA,
            pltpu.SemaphoreType.DMA,
        ],
    )
    return f(idx4d, table)


def kernel(tokens, table):
    bsz, seq = tokens.shape
    na, nb = seq // _SUB, bsz // _LANE
    tv = (jnp.transpose(tokens.astype(jnp.int32))
          .reshape(na, _SUB, nb, _LANE).transpose(0, 2, 1, 3))
    lin = _gather(tv, table)
    return lin.transpose(2, 4, 0, 1, 3).reshape(bsz, seq, _EMB)
